# baseline (speedup 1.0000x reference)
import functools

import jax
import jax.numpy as jnp
from jax import lax
from jax.experimental import pallas as pl
from jax.experimental.pallas import tpu as pltpu
from jax.experimental.pallas import tpu_sc as plsc

B = 4
N_OBJ = 31
N_ALL = N_OBJ + 1
Z_WHAT = 64
IMG = 224
OBJ = 64
HID = 1024
OUTD = 3 * OBJ * OBJ
NB = 1536
HALF = (OBJ - 1) / 2.0
BANDS = 8
BROWS = IMG // BANDS
XBLK = IMG // 16


def _decode_body(z_ref, W1_ref, b1_ref, W2_ref, b2_ref, out_ref, h_ref):
    j = pl.program_id(0)

    @pl.when(j == 0)
    def _():
        h = jnp.dot(z_ref[...], W1_ref[...], preferred_element_type=jnp.float32)
        h_ref[...] = jnp.maximum(h + b1_ref[...], 0.0)

    y = jnp.dot(h_ref[...], W2_ref[...], preferred_element_type=jnp.float32)
    out_ref[...] = jax.nn.sigmoid(y + b2_ref[...])


def _axis_tables(lin, ctr, scale):
    s = jnp.maximum(scale, 1e-6)
    p = ((lin - (2.0 * ctr - 1.0)) / s + 1.0) * HALF
    p0 = jnp.floor(p)
    w = p - p0
    v0 = (p0 >= 0) & (p0 <= OBJ - 1)
    v1 = (p0 + 1.0 >= 0) & (p0 + 1.0 <= OBJ - 1)
    i0 = jnp.clip(p0, 0, OBJ - 1).astype(jnp.int32)
    i1 = jnp.clip(p0 + 1.0, 0, OBJ - 1).astype(jnp.int32)
    w0 = jnp.where(v0, 1.0 - w, 0.0)
    w1 = jnp.where(v1, w, 0.0)
    return i0, i1, w0, w1


def _prep_body(lin_ref, zw_ref, ld_ref,
               x0_ref, x1_ref, wx0_ref, wx1_ref,
               y0_ref, y1_ref, wy0_ref, wy1_ref,
               w_ref, ylo_ref, yhi_ref, xblo_ref, xbhi_ref):
    lin = lin_ref[...]
    cx = zw_ref[:, :, 0:1]
    cy = zw_ref[:, :, 1:2]
    sx = zw_ref[:, :, 2:3]
    sy = zw_ref[:, :, 3:4]

    x0, x1, wx0, wx1 = _axis_tables(lin, cx, sx)
    y0, y1, wy0, wy1 = _axis_tables(lin, cy, sy)
    x0_ref[...] = x0
    x1_ref[...] = x1
    wx0_ref[...] = wx0
    wx1_ref[...] = wx1
    y0_ref[...] = y0 * OBJ
    y1_ref[...] = y1 * OBJ
    wy0_ref[...] = wy0
    wy1_ref[...] = wy1

    ld = ld_ref[...]
    m = jnp.max(ld, axis=1, keepdims=True)
    e = jnp.exp(ld - m)
    s = jnp.sum(e, axis=1, keepdims=True)
    wmat = e / s
    col = jax.lax.broadcasted_iota(jnp.int32, (B, N_ALL), 1)
    wmat = jnp.where(col == N_OBJ, 1.0, wmat)
    w_ref[...] = wmat

    t = jax.lax.broadcasted_iota(jnp.int32, (B, N_ALL, IMG), 2)
    vy = (wy0 != 0.0) | (wy1 != 0.0)
    ylo = jnp.min(jnp.where(vy, t, IMG), axis=2)
    yhi = jnp.max(jnp.where(vy, t + 1, 0), axis=2)
    ylo_ref[...] = jnp.where(wmat > 0.0, ylo, IMG)
    yhi_ref[...] = jnp.where(wmat > 0.0, yhi, 0)
    vx = (wx0 != 0.0) | (wx1 != 0.0)
    xlo = jnp.min(jnp.where(vx, t, IMG), axis=2)
    xhi = jnp.max(jnp.where(vx, t + 1, 0), axis=2)
    xblo_ref[...] = xlo // 16
    xbhi_ref[...] = (xhi + 15) // 16


_PATCH = OBJ * OBJ
_CHAN = BROWS * IMG


def _sc_merge_body(d3, x0_h, x1_h, wx0_h, wx1_h, y0_h, y1_h, wy0_h, wy1_h,
                   meta_h, w_h, zz_h, out_h,
                   x0_v, x1_v, wx0_v, wx1_v, y0_v, y1_v, wy0_v, wy1_v,
                   meta_v, w_v, patch_v, acc_v, bg_v, sem0, sem1):
    cc = lax.axis_index("c")
    ss = lax.axis_index("s")
    wid = ss * 2 + cc
    b = wid // BANDS
    band = wid % BANDS
    blo = band * BROWS
    row0 = b * N_ALL

    t0 = row0 * IMG
    for hsrc, vdst in ((x0_h, x0_v), (x1_h, x1_v), (wx0_h, wx0_v),
                       (wx1_h, wx1_v), (y0_h, y0_v), (y1_h, y1_v),
                       (wy0_h, wy0_v), (wy1_h, wy1_v)):
        pltpu.sync_copy(hsrc.at[pl.ds(t0, N_ALL * IMG)], vdst)
    pltpu.sync_copy(meta_h.at[pl.ds(row0 * 16, N_ALL * 16)], meta_v)
    pltpu.sync_copy(w_h.at[pl.ds(row0 * 16, N_ALL * 16)], w_v)
    pltpu.sync_copy(zz_h, acc_v)
    pltpu.sync_copy(zz_h, bg_v)

    def spl(v):
        return jnp.full((16,), v, jnp.int32)

    PB = 3 * _PATCH

    def start_fetch(i, buf, sem):
        pltpu.make_async_copy(
            d3.at[pl.ds((row0 + i) * PB, PB)],
            patch_v.at[pl.ds(buf * PB, PB)], sem).start()

    def wait_fetch(buf, sem):
        pltpu.make_async_copy(
            d3.at[pl.ds(0, PB)],
            patch_v.at[pl.ds(buf * PB, PB)], sem).wait()

    def do_object(i, boff, tgt):
        mrow = meta_v[pl.ds(i * 16, 16)]
        ylo = jnp.maximum(mrow[0], blo)
        yhi = jnp.minimum(mrow[1], blo + BROWS)
        xblo = mrow[2]
        xbhi = mrow[3]
        ws = w_v[pl.ds(i * 16, 16)][0]
        toff = i * IMG

        @pl.when(ylo < yhi)
        def _():
            @plsc.parallel_loop(ylo, yhi)
            def row_body(y):
                sy = spl(toff + y)
                y0m = plsc.load_gather(y0_v, [sy])
                y1m = plsc.load_gather(y1_v, [sy])
                wy0 = plsc.load_gather(wy0_v, [sy]) * ws
                wy1 = plsc.load_gather(wy1_v, [sy]) * ws
                aoff0 = (y - blo) * IMG

                @plsc.parallel_loop(xblo, xbhi, unroll=4)
                def col_body(xb):
                    bs = xb * 16
                    x0v = x0_v[pl.ds(toff + bs, 16)]
                    x1v = x1_v[pl.ds(toff + bs, 16)]
                    wx0v = wx0_v[pl.ds(toff + bs, 16)]
                    wx1v = wx1_v[pl.ds(toff + bs, 16)]
                    for c in range(3):
                        co = boff + c * _PATCH
                        g00 = plsc.load_gather(patch_v, [y0m + x0v + co])
                        g01 = plsc.load_gather(patch_v, [y0m + x1v + co])
                        g10 = plsc.load_gather(patch_v, [y1m + x0v + co])
                        g11 = plsc.load_gather(patch_v, [y1m + x1v + co])
                        r = (wy0 * (wx0v * g00 + wx1v * g01)
                             + wy1 * (wx0v * g10 + wx1v * g11))
                        aoff = c * _CHAN + aoff0 + bs
                        cur = tgt[pl.ds(aoff, 16)]
                        tgt[pl.ds(aoff, 16)] = cur + r

    start_fetch(0, 0, sem0)
    start_fetch(1, 1, sem1)

    def pair_body(j, _):
        i0 = j * 2
        wait_fetch(0, sem0)
        do_object(i0, 0, acc_v)

        @pl.when(i0 + 2 < N_ALL)
        def _():
            start_fetch(i0 + 2, 0, sem0)

        wait_fetch(1, sem1)
        do_object(i0 + 1, PB, acc_v)

        @pl.when(i0 + 3 < N_ALL)
        def _():
            start_fetch(i0 + 3, 1, sem1)

        return 0

    lax.fori_loop(0, N_ALL // 2 - 1, pair_body, 0)
    wait_fetch(0, sem0)
    do_object(N_ALL - 2, 0, acc_v)
    wait_fetch(1, sem1)
    do_object(N_ALL - 1, PB, bg_v)

    @plsc.parallel_loop(0, 3 * _CHAN // 16, unroll=4)
    def crow(j):
        off = j * 16
        a = acc_v[pl.ds(off, 16)]
        g = bg_v[pl.ds(off, 16)]
        acc_v[pl.ds(off, 16)] = a + g * jnp.where(a < 0.001, 1.0, 0.0)
    for c in range(3):
        pltpu.sync_copy(acc_v.at[pl.ds(c * _CHAN, _CHAN)], out_h.at[b, c, band])


@jax.jit
def kernel(z_what, z_where, z_present, z_depth, W1, b1, W2, b2):
    zf = z_what.reshape(B * N_ALL, Z_WHAT)

    decoded = pl.pallas_call(
        _decode_body,
        grid=(OUTD // NB,),
        in_specs=[
            pl.BlockSpec((B * N_ALL, Z_WHAT), lambda j: (0, 0)),
            pl.BlockSpec((Z_WHAT, HID), lambda j: (0, 0)),
            pl.BlockSpec((1, HID), lambda j: (0, 0)),
            pl.BlockSpec((HID, NB), lambda j: (0, j)),
            pl.BlockSpec((1, NB), lambda j: (0, j)),
        ],
        out_specs=pl.BlockSpec((B * N_ALL, NB), lambda j: (0, j)),
        out_shape=jax.ShapeDtypeStruct((B * N_ALL, OUTD), jnp.float32),
        scratch_shapes=[pltpu.VMEM((B * N_ALL, HID), jnp.float32)],
    )(zf, W1, b1.reshape(1, HID), W2, b2.reshape(1, OUTD))

    bg_where = jnp.broadcast_to(
        jnp.array([0.5, 0.5, 1.0, 1.0], jnp.float32), (B, 1, 4))
    zw_f = jnp.concatenate([z_where, bg_where], axis=1)

    neg_inf = jnp.full((B, 1), -jnp.inf, jnp.float32)
    dcol = jnp.concatenate([z_depth[..., 0], neg_inf], axis=1)
    pcol = jnp.concatenate([z_present[..., 0], jnp.zeros((B, 1))], axis=1)
    ld = jnp.where(pcol == 1.0, dcol, -jnp.inf)

    lin = jnp.linspace(-1.0, 1.0, IMG).astype(jnp.float32).reshape(1, 1, IMG)

    tab3 = jax.ShapeDtypeStruct((B, N_ALL, IMG), jnp.float32)
    tab3i = jax.ShapeDtypeStruct((B, N_ALL, IMG), jnp.int32)
    tab2 = jax.ShapeDtypeStruct((B, N_ALL), jnp.float32)
    tab2i = jax.ShapeDtypeStruct((B, N_ALL), jnp.int32)

    full = lambda s: pl.BlockSpec(s, lambda: tuple(0 for _ in s))
    (x0, x1, wx0, wx1, y0, y1, wy0, wy1,
     wmat, ylo, yhi, xblo, xbhi) = pl.pallas_call(
        _prep_body,
        in_specs=[full((1, 1, IMG)), full((B, N_ALL, 4)), full((B, N_ALL))],
        out_specs=[full((B, N_ALL, IMG))] * 8 + [full((B, N_ALL))] * 5,
        out_shape=[tab3i, tab3i, tab3, tab3, tab3i, tab3i, tab3, tab3,
                   tab2, tab2i, tab2i, tab2i, tab2i],
    )(lin, zw_f, ld)

    M = B * N_ALL
    MI = M * IMG
    zpad = jnp.zeros((B, N_ALL), jnp.int32)
    meta = jnp.stack([ylo, yhi, xblo, xbhi] + [zpad] * 12, axis=2)
    meta = meta.reshape(M * 16)
    wv = jnp.broadcast_to(wmat.reshape(M, 1), (M, 16)).reshape(M * 16)
    zz = jnp.zeros((3 * _CHAN,), jnp.float32)
    d3f = decoded.reshape(M * 3 * _PATCH)

    tabs = [t.reshape(MI) for t in (x0, x1, wx0, wx1, y0, y1, wy0, wy1)]

    mesh = plsc.VectorSubcoreMesh(core_axis_name="c", subcore_axis_name="s")
    out5 = pl.kernel(
        _sc_merge_body,
        out_type=jax.ShapeDtypeStruct((B, 3, BANDS, _CHAN), jnp.float32),
        mesh=mesh,
        scratch_types=(
            [pltpu.VMEM((MI // B,), jnp.int32)] * 2
            + [pltpu.VMEM((MI // B,), jnp.float32)] * 2
            + [pltpu.VMEM((MI // B,), jnp.int32)] * 2
            + [pltpu.VMEM((MI // B,), jnp.float32)] * 2
            + [pltpu.VMEM((N_ALL * 16,), jnp.int32),
               pltpu.VMEM((N_ALL * 16,), jnp.float32),
               pltpu.VMEM((2 * 3 * _PATCH,), jnp.float32),
               pltpu.VMEM((3 * _CHAN,), jnp.float32),
               pltpu.VMEM((3 * _CHAN,), jnp.float32),
               pltpu.SemaphoreType.DMA,
               pltpu.SemaphoreType.DMA]
        ),
        compiler_params=pltpu.CompilerParams(needs_layout_passes=False),
    )(d3f, *tabs, meta, wv, zz)

    return out5.reshape(B, 3, IMG, IMG)


# speedup vs baseline: 1.0023x; 1.0023x over previous
import functools

import jax
import jax.numpy as jnp
from jax import lax
from jax.experimental import pallas as pl
from jax.experimental.pallas import tpu as pltpu
from jax.experimental.pallas import tpu_sc as plsc

B = 4
N_OBJ = 31
N_ALL = N_OBJ + 1
Z_WHAT = 64
IMG = 224
OBJ = 64
HID = 1024
OUTD = 3 * OBJ * OBJ
NB = 1536
HALF = (OBJ - 1) / 2.0
BANDS = 8
BROWS = IMG // BANDS
XBLK = IMG // 16


def _decode_body(z_ref, W1_ref, b1_ref, W2_ref, b2_ref, out_ref, h_ref):
    j = pl.program_id(0)

    @pl.when(j == 0)
    def _():
        h = jnp.dot(z_ref[...], W1_ref[...], preferred_element_type=jnp.float32)
        h_ref[...] = jnp.maximum(h + b1_ref[...], 0.0)

    y = jnp.dot(h_ref[...].astype(jnp.bfloat16),
                W2_ref[...].astype(jnp.bfloat16),
                preferred_element_type=jnp.float32)
    out_ref[...] = jax.nn.sigmoid(y + b2_ref[...])


def _axis_tables(lin, ctr, scale):
    s = jnp.maximum(scale, 1e-6)
    p = ((lin - (2.0 * ctr - 1.0)) / s + 1.0) * HALF
    p0 = jnp.floor(p)
    w = p - p0
    v0 = (p0 >= 0) & (p0 <= OBJ - 1)
    v1 = (p0 + 1.0 >= 0) & (p0 + 1.0 <= OBJ - 1)
    i0 = jnp.clip(p0, 0, OBJ - 1).astype(jnp.int32)
    i1 = jnp.clip(p0 + 1.0, 0, OBJ - 1).astype(jnp.int32)
    w0 = jnp.where(v0, 1.0 - w, 0.0)
    w1 = jnp.where(v1, w, 0.0)
    return i0, i1, w0, w1


def _prep_body(lin_ref, zw_ref, ld_ref,
               x0_ref, x1_ref, wx0_ref, wx1_ref,
               y0_ref, y1_ref, wy0_ref, wy1_ref,
               w_ref, ylo_ref, yhi_ref, xblo_ref, xbhi_ref):
    lin = lin_ref[...]
    cx = zw_ref[:, :, 0:1]
    cy = zw_ref[:, :, 1:2]
    sx = zw_ref[:, :, 2:3]
    sy = zw_ref[:, :, 3:4]

    x0, x1, wx0, wx1 = _axis_tables(lin, cx, sx)
    y0, y1, wy0, wy1 = _axis_tables(lin, cy, sy)
    x0_ref[...] = x0
    x1_ref[...] = x1
    wx0_ref[...] = wx0
    wx1_ref[...] = wx1
    y0_ref[...] = y0 * OBJ
    y1_ref[...] = y1 * OBJ
    wy0_ref[...] = wy0
    wy1_ref[...] = wy1

    ld = ld_ref[...]
    m = jnp.max(ld, axis=1, keepdims=True)
    e = jnp.exp(ld - m)
    s = jnp.sum(e, axis=1, keepdims=True)
    wmat = e / s
    col = jax.lax.broadcasted_iota(jnp.int32, (B, N_ALL), 1)
    wmat = jnp.where(col == N_OBJ, 1.0, wmat)
    w_ref[...] = wmat

    t = jax.lax.broadcasted_iota(jnp.int32, (B, N_ALL, IMG), 2)
    vy = (wy0 != 0.0) | (wy1 != 0.0)
    ylo = jnp.min(jnp.where(vy, t, IMG), axis=2)
    yhi = jnp.max(jnp.where(vy, t + 1, 0), axis=2)
    ylo_ref[...] = jnp.where(wmat > 0.0, ylo, IMG)
    yhi_ref[...] = jnp.where(wmat > 0.0, yhi, 0)
    vx = (wx0 != 0.0) | (wx1 != 0.0)
    xlo = jnp.min(jnp.where(vx, t, IMG), axis=2)
    xhi = jnp.max(jnp.where(vx, t + 1, 0), axis=2)
    xblo_ref[...] = xlo // 16
    xbhi_ref[...] = (xhi + 15) // 16


_PATCH = OBJ * OBJ
_CHAN = BROWS * IMG


def _sc_merge_body(d3, x0_h, x1_h, wx0_h, wx1_h, y0_h, y1_h, wy0_h, wy1_h,
                   meta_h, w_h, zz_h, out_h,
                   x0_v, x1_v, wx0_v, wx1_v, y0_v, y1_v, wy0_v, wy1_v,
                   meta_v, w_v, patch_v, acc_v, bg_v, sem0, sem1):
    cc = lax.axis_index("c")
    ss = lax.axis_index("s")
    wid = ss * 2 + cc
    b = wid // BANDS
    band = wid % BANDS
    blo = band * BROWS
    row0 = b * N_ALL

    t0 = row0 * IMG
    for hsrc, vdst in ((x0_h, x0_v), (x1_h, x1_v), (wx0_h, wx0_v),
                       (wx1_h, wx1_v), (y0_h, y0_v), (y1_h, y1_v),
                       (wy0_h, wy0_v), (wy1_h, wy1_v)):
        pltpu.sync_copy(hsrc.at[pl.ds(t0, N_ALL * IMG)], vdst)
    pltpu.sync_copy(meta_h.at[pl.ds(row0 * 16, N_ALL * 16)], meta_v)
    pltpu.sync_copy(w_h.at[pl.ds(row0 * 16, N_ALL * 16)], w_v)
    pltpu.sync_copy(zz_h, acc_v)
    pltpu.sync_copy(zz_h, bg_v)

    def spl(v):
        return jnp.full((16,), v, jnp.int32)

    PB = 3 * _PATCH

    def start_fetch(i, buf, sem):
        pltpu.make_async_copy(
            d3.at[pl.ds((row0 + i) * PB, PB)],
            patch_v.at[pl.ds(buf * PB, PB)], sem).start()

    def wait_fetch(buf, sem):
        pltpu.make_async_copy(
            d3.at[pl.ds(0, PB)],
            patch_v.at[pl.ds(buf * PB, PB)], sem).wait()

    def do_object(i, boff, tgt):
        mrow = meta_v[pl.ds(i * 16, 16)]
        ylo = jnp.maximum(mrow[0], blo)
        yhi = jnp.minimum(mrow[1], blo + BROWS)
        xblo = mrow[2]
        xbhi = mrow[3]
        ws = w_v[pl.ds(i * 16, 16)][0]
        toff = i * IMG

        @pl.when(ylo < yhi)
        def _():
            @plsc.parallel_loop(ylo, yhi)
            def row_body(y):
                sy = spl(toff + y)
                y0m = plsc.load_gather(y0_v, [sy])
                y1m = plsc.load_gather(y1_v, [sy])
                wy0 = plsc.load_gather(wy0_v, [sy]) * ws
                wy1 = plsc.load_gather(wy1_v, [sy]) * ws
                aoff0 = (y - blo) * IMG

                @plsc.parallel_loop(xblo, xbhi, unroll=4)
                def col_body(xb):
                    bs = xb * 16
                    x0v = x0_v[pl.ds(toff + bs, 16)]
                    x1v = x1_v[pl.ds(toff + bs, 16)]
                    wx0v = wx0_v[pl.ds(toff + bs, 16)]
                    wx1v = wx1_v[pl.ds(toff + bs, 16)]
                    for c in range(3):
                        co = boff + c * _PATCH
                        g00 = plsc.load_gather(patch_v, [y0m + x0v + co])
                        g01 = plsc.load_gather(patch_v, [y0m + x1v + co])
                        g10 = plsc.load_gather(patch_v, [y1m + x0v + co])
                        g11 = plsc.load_gather(patch_v, [y1m + x1v + co])
                        r = (wy0 * (wx0v * g00 + wx1v * g01)
                             + wy1 * (wx0v * g10 + wx1v * g11))
                        aoff = c * _CHAN + aoff0 + bs
                        cur = tgt[pl.ds(aoff, 16)]
                        tgt[pl.ds(aoff, 16)] = cur + r

    start_fetch(0, 0, sem0)
    start_fetch(1, 1, sem1)

    def pair_body(j, _):
        i0 = j * 2
        wait_fetch(0, sem0)
        do_object(i0, 0, acc_v)

        @pl.when(i0 + 2 < N_ALL)
        def _():
            start_fetch(i0 + 2, 0, sem0)

        wait_fetch(1, sem1)
        do_object(i0 + 1, PB, acc_v)

        @pl.when(i0 + 3 < N_ALL)
        def _():
            start_fetch(i0 + 3, 1, sem1)

        return 0

    lax.fori_loop(0, N_ALL // 2 - 1, pair_body, 0)
    wait_fetch(0, sem0)
    do_object(N_ALL - 2, 0, acc_v)
    wait_fetch(1, sem1)
    do_object(N_ALL - 1, PB, bg_v)

    @plsc.parallel_loop(0, 3 * _CHAN // 16, unroll=4)
    def crow(j):
        off = j * 16
        a = acc_v[pl.ds(off, 16)]
        g = bg_v[pl.ds(off, 16)]
        acc_v[pl.ds(off, 16)] = a + g * jnp.where(a < 0.001, 1.0, 0.0)
    for c in range(3):
        pltpu.sync_copy(acc_v.at[pl.ds(c * _CHAN, _CHAN)], out_h.at[b, c, band])


@jax.jit
def kernel(z_what, z_where, z_present, z_depth, W1, b1, W2, b2):
    zf = z_what.reshape(B * N_ALL, Z_WHAT)

    decoded = pl.pallas_call(
        _decode_body,
        grid=(OUTD // NB,),
        in_specs=[
            pl.BlockSpec((B * N_ALL, Z_WHAT), lambda j: (0, 0)),
            pl.BlockSpec((Z_WHAT, HID), lambda j: (0, 0)),
            pl.BlockSpec((1, HID), lambda j: (0, 0)),
            pl.BlockSpec((HID, NB), lambda j: (0, j)),
            pl.BlockSpec((1, NB), lambda j: (0, j)),
        ],
        out_specs=pl.BlockSpec((B * N_ALL, NB), lambda j: (0, j)),
        out_shape=jax.ShapeDtypeStruct((B * N_ALL, OUTD), jnp.float32),
        scratch_shapes=[pltpu.VMEM((B * N_ALL, HID), jnp.float32)],
    )(zf, W1, b1.reshape(1, HID), W2, b2.reshape(1, OUTD))

    bg_where = jnp.broadcast_to(
        jnp.array([0.5, 0.5, 1.0, 1.0], jnp.float32), (B, 1, 4))
    zw_f = jnp.concatenate([z_where, bg_where], axis=1)

    neg_inf = jnp.full((B, 1), -jnp.inf, jnp.float32)
    dcol = jnp.concatenate([z_depth[..., 0], neg_inf], axis=1)
    pcol = jnp.concatenate([z_present[..., 0], jnp.zeros((B, 1))], axis=1)
    ld = jnp.where(pcol == 1.0, dcol, -jnp.inf)

    lin = jnp.linspace(-1.0, 1.0, IMG).astype(jnp.float32).reshape(1, 1, IMG)

    tab3 = jax.ShapeDtypeStruct((B, N_ALL, IMG), jnp.float32)
    tab3i = jax.ShapeDtypeStruct((B, N_ALL, IMG), jnp.int32)
    tab2 = jax.ShapeDtypeStruct((B, N_ALL), jnp.float32)
    tab2i = jax.ShapeDtypeStruct((B, N_ALL), jnp.int32)

    full = lambda s: pl.BlockSpec(s, lambda: tuple(0 for _ in s))
    (x0, x1, wx0, wx1, y0, y1, wy0, wy1,
     wmat, ylo, yhi, xblo, xbhi) = pl.pallas_call(
        _prep_body,
        in_specs=[full((1, 1, IMG)), full((B, N_ALL, 4)), full((B, N_ALL))],
        out_specs=[full((B, N_ALL, IMG))] * 8 + [full((B, N_ALL))] * 5,
        out_shape=[tab3i, tab3i, tab3, tab3, tab3i, tab3i, tab3, tab3,
                   tab2, tab2i, tab2i, tab2i, tab2i],
    )(lin, zw_f, ld)

    M = B * N_ALL
    MI = M * IMG
    zpad = jnp.zeros((B, N_ALL), jnp.int32)
    meta = jnp.stack([ylo, yhi, xblo, xbhi] + [zpad] * 12, axis=2)
    meta = meta.reshape(M * 16)
    wv = jnp.broadcast_to(wmat.reshape(M, 1), (M, 16)).reshape(M * 16)
    zz = jnp.zeros((3 * _CHAN,), jnp.float32)
    d3f = decoded.reshape(M * 3 * _PATCH)

    tabs = [t.reshape(MI) for t in (x0, x1, wx0, wx1, y0, y1, wy0, wy1)]

    mesh = plsc.VectorSubcoreMesh(core_axis_name="c", subcore_axis_name="s")
    out5 = pl.kernel(
        _sc_merge_body,
        out_type=jax.ShapeDtypeStruct((B, 3, BANDS, _CHAN), jnp.float32),
        mesh=mesh,
        scratch_types=(
            [pltpu.VMEM((MI // B,), jnp.int32)] * 2
            + [pltpu.VMEM((MI // B,), jnp.float32)] * 2
            + [pltpu.VMEM((MI // B,), jnp.int32)] * 2
            + [pltpu.VMEM((MI // B,), jnp.float32)] * 2
            + [pltpu.VMEM((N_ALL * 16,), jnp.int32),
               pltpu.VMEM((N_ALL * 16,), jnp.float32),
               pltpu.VMEM((2 * 3 * _PATCH,), jnp.float32),
               pltpu.VMEM((3 * _CHAN,), jnp.float32),
               pltpu.VMEM((3 * _CHAN,), jnp.float32),
               pltpu.SemaphoreType.DMA,
               pltpu.SemaphoreType.DMA]
        ),
        compiler_params=pltpu.CompilerParams(needs_layout_passes=False),
    )(d3f, *tabs, meta, wv, zz)

    return out5.reshape(B, 3, IMG, IMG)


# speedup vs baseline: 1.0727x; 1.0702x over previous
import functools

import jax
import jax.numpy as jnp
from jax import lax
from jax.experimental import pallas as pl
from jax.experimental.pallas import tpu as pltpu
from jax.experimental.pallas import tpu_sc as plsc

B = 4
N_OBJ = 31
N_ALL = N_OBJ + 1
Z_WHAT = 64
IMG = 224
OBJ = 64
HID = 1024
OUTD = 3 * OBJ * OBJ
NB = 1536
HALF = (OBJ - 1) / 2.0
BANDS = 8
BROWS = IMG // BANDS
XBLK = IMG // 16


def _axis_tables(lin, ctr, scale):
    s = jnp.maximum(scale, 1e-6)
    p = ((lin - (2.0 * ctr - 1.0)) / s + 1.0) * HALF
    p0 = jnp.floor(p)
    w = p - p0
    v0 = (p0 >= 0) & (p0 <= OBJ - 1)
    v1 = (p0 + 1.0 >= 0) & (p0 + 1.0 <= OBJ - 1)
    i0 = jnp.clip(p0, 0, OBJ - 1).astype(jnp.int32)
    i1 = jnp.clip(p0 + 1.0, 0, OBJ - 1).astype(jnp.int32)
    w0 = jnp.where(v0, 1.0 - w, 0.0)
    w1 = jnp.where(v1, w, 0.0)
    return i0, i1, w0, w1


def _decode_body(z_ref, W1_ref, b1_ref, W2_ref, b2_ref, lin_ref, zw_ref,
                 ld_ref, out_ref,
                 x0_ref, x1_ref, wx0_ref, wx1_ref,
                 y0_ref, y1_ref, wy0_ref, wy1_ref,
                 w_ref, ylo_ref, yhi_ref, xblo_ref, xbhi_ref, h_ref):
    j = pl.program_id(0)

    @pl.when(j == 0)
    def _():
        h = jnp.dot(z_ref[...], W1_ref[...], preferred_element_type=jnp.float32)
        h_ref[...] = jnp.maximum(h + b1_ref[...], 0.0)

        lin = lin_ref[...]
        x0, x1, wx0, wx1 = _axis_tables(lin, zw_ref[:, 0:1], zw_ref[:, 2:3])
        y0, y1, wy0, wy1 = _axis_tables(lin, zw_ref[:, 1:2], zw_ref[:, 3:4])
        x0_ref[...] = x0
        x1_ref[...] = x1
        wx0_ref[...] = wx0
        wx1_ref[...] = wx1
        y0_ref[...] = y0 * OBJ
        y1_ref[...] = y1 * OBJ
        wy0_ref[...] = wy0
        wy1_ref[...] = wy1

        ld = ld_ref[...]
        m = jnp.max(ld, axis=1, keepdims=True)
        e = jnp.exp(ld - m)
        s = jnp.sum(e, axis=1, keepdims=True)
        wmat = e / s
        col = jax.lax.broadcasted_iota(jnp.int32, (B, N_ALL), 1)
        w_ref[...] = jnp.where(col == N_OBJ, 1.0, wmat)

        t = jax.lax.broadcasted_iota(jnp.int32, (B * N_ALL, IMG), 1)
        vy = (wy0 != 0.0) | (wy1 != 0.0)
        ylo_ref[...] = jnp.min(jnp.where(vy, t, IMG), axis=1, keepdims=True)
        yhi_ref[...] = jnp.max(jnp.where(vy, t + 1, 0), axis=1, keepdims=True)
        vx = (wx0 != 0.0) | (wx1 != 0.0)
        xlo = jnp.min(jnp.where(vx, t, IMG), axis=1, keepdims=True)
        xhi = jnp.max(jnp.where(vx, t + 1, 0), axis=1, keepdims=True)
        xblo_ref[...] = xlo // 16
        xbhi_ref[...] = (xhi + 15) // 16

    y = jnp.dot(h_ref[...], W2_ref[...], preferred_element_type=jnp.float32)
    out_ref[...] = jax.nn.sigmoid(y + b2_ref[...])


_PATCH = OBJ * OBJ
_CHAN = BROWS * IMG


def _sc_merge_body(d3, x0_h, x1_h, wx0_h, wx1_h, y0_h, y1_h, wy0_h, wy1_h,
                   meta_h, w_h, zz_h, out_h,
                   x0_v, x1_v, wx0_v, wx1_v, y0_v, y1_v, wy0_v, wy1_v,
                   meta_v, w_v, patch_v, acc_v, bg_v, sem0, sem1):
    cc = lax.axis_index("c")
    ss = lax.axis_index("s")
    wid = ss * 2 + cc
    b = wid // BANDS
    band = wid % BANDS
    blo = band * BROWS
    row0 = b * N_ALL

    t0 = row0 * IMG
    for hsrc, vdst in ((x0_h, x0_v), (x1_h, x1_v), (wx0_h, wx0_v),
                       (wx1_h, wx1_v), (y0_h, y0_v), (y1_h, y1_v),
                       (wy0_h, wy0_v), (wy1_h, wy1_v)):
        pltpu.sync_copy(hsrc.at[pl.ds(t0, N_ALL * IMG)], vdst)
    pltpu.sync_copy(meta_h.at[pl.ds(row0 * 16, N_ALL * 16)], meta_v)
    pltpu.sync_copy(w_h.at[pl.ds(row0 * 16, N_ALL * 16)], w_v)
    pltpu.sync_copy(zz_h, acc_v)
    pltpu.sync_copy(zz_h, bg_v)

    def spl(v):
        return jnp.full((16,), v, jnp.int32)

    PB = 3 * _PATCH

    def start_fetch(i, buf, sem):
        pltpu.make_async_copy(
            d3.at[pl.ds((row0 + i) * PB, PB)],
            patch_v.at[pl.ds(buf * PB, PB)], sem).start()

    def wait_fetch(buf, sem):
        pltpu.make_async_copy(
            d3.at[pl.ds(0, PB)],
            patch_v.at[pl.ds(buf * PB, PB)], sem).wait()

    def do_object(i, boff, tgt):
        mrow = meta_v[pl.ds(i * 16, 16)]
        ylo = jnp.maximum(mrow[0], blo)
        yhi = jnp.minimum(mrow[1], blo + BROWS)
        xblo = mrow[2]
        xbhi = mrow[3]
        ws = w_v[pl.ds(i * 16, 16)][0]
        toff = i * IMG

        @pl.when((ylo < yhi) & (ws != 0.0))
        def _():
            @plsc.parallel_loop(ylo, yhi)
            def row_body(y):
                sy = spl(toff + y)
                y0m = plsc.load_gather(y0_v, [sy])
                y1m = plsc.load_gather(y1_v, [sy])
                wy0 = plsc.load_gather(wy0_v, [sy]) * ws
                wy1 = plsc.load_gather(wy1_v, [sy]) * ws
                aoff0 = (y - blo) * IMG

                @plsc.parallel_loop(xblo, xbhi, unroll=4)
                def col_body(xb):
                    bs = xb * 16
                    x0v = x0_v[pl.ds(toff + bs, 16)]
                    x1v = x1_v[pl.ds(toff + bs, 16)]
                    wx0v = wx0_v[pl.ds(toff + bs, 16)]
                    wx1v = wx1_v[pl.ds(toff + bs, 16)]
                    for c in range(3):
                        co = boff + c * _PATCH
                        g00 = plsc.load_gather(patch_v, [y0m + x0v + co])
                        g01 = plsc.load_gather(patch_v, [y0m + x1v + co])
                        g10 = plsc.load_gather(patch_v, [y1m + x0v + co])
                        g11 = plsc.load_gather(patch_v, [y1m + x1v + co])
                        r = (wy0 * (wx0v * g00 + wx1v * g01)
                             + wy1 * (wx0v * g10 + wx1v * g11))
                        aoff = c * _CHAN + aoff0 + bs
                        cur = tgt[pl.ds(aoff, 16)]
                        tgt[pl.ds(aoff, 16)] = cur + r

    start_fetch(0, 0, sem0)
    start_fetch(1, 1, sem1)

    def pair_body(j, _):
        i0 = j * 2
        wait_fetch(0, sem0)
        do_object(i0, 0, acc_v)

        @pl.when(i0 + 2 < N_ALL)
        def _():
            start_fetch(i0 + 2, 0, sem0)

        wait_fetch(1, sem1)
        do_object(i0 + 1, PB, acc_v)

        @pl.when(i0 + 3 < N_ALL)
        def _():
            start_fetch(i0 + 3, 1, sem1)

        return 0

    lax.fori_loop(0, N_ALL // 2 - 1, pair_body, 0)
    wait_fetch(0, sem0)
    do_object(N_ALL - 2, 0, acc_v)
    wait_fetch(1, sem1)
    do_object(N_ALL - 1, PB, bg_v)

    @plsc.parallel_loop(0, 3 * _CHAN // 16, unroll=4)
    def crow(j):
        off = j * 16
        a = acc_v[pl.ds(off, 16)]
        g = bg_v[pl.ds(off, 16)]
        acc_v[pl.ds(off, 16)] = a + g * jnp.where(a < 0.001, 1.0, 0.0)
    for c in range(3):
        pltpu.sync_copy(acc_v.at[pl.ds(c * _CHAN, _CHAN)], out_h.at[b, c, band])


@jax.jit
def kernel(z_what, z_where, z_present, z_depth, W1, b1, W2, b2):
    zf = z_what.reshape(B * N_ALL, Z_WHAT)

    M = B * N_ALL
    MI = M * IMG

    bg_where = jnp.broadcast_to(
        jnp.array([0.5, 0.5, 1.0, 1.0], jnp.float32), (B, 1, 4))
    zw_f = jnp.concatenate([z_where, bg_where], axis=1).reshape(M, 4)

    neg_inf = jnp.full((B, 1), -jnp.inf, jnp.float32)
    dcol = jnp.concatenate([z_depth[..., 0], neg_inf], axis=1)
    pcol = jnp.concatenate([z_present[..., 0], jnp.zeros((B, 1))], axis=1)
    ld = jnp.where(pcol == 1.0, dcol, -jnp.inf)

    lin = jnp.linspace(-1.0, 1.0, IMG).astype(jnp.float32).reshape(1, IMG)

    tabf = jax.ShapeDtypeStruct((M, IMG), jnp.float32)
    tabi = jax.ShapeDtypeStruct((M, IMG), jnp.int32)
    coli = jax.ShapeDtypeStruct((M, 1), jnp.int32)

    cst = lambda s: pl.BlockSpec(s, lambda j: tuple(0 for _ in s))
    (decoded, x0, x1, wx0, wx1, y0, y1, wy0, wy1,
     wmat, ylo, yhi, xblo, xbhi) = pl.pallas_call(
        _decode_body,
        grid=(OUTD // NB,),
        in_specs=[
            cst((M, Z_WHAT)),
            cst((Z_WHAT, HID)),
            cst((1, HID)),
            pl.BlockSpec((HID, NB), lambda j: (0, j)),
            pl.BlockSpec((1, NB), lambda j: (0, j)),
            cst((1, IMG)),
            cst((M, 4)),
            cst((B, N_ALL)),
        ],
        out_specs=([pl.BlockSpec((M, NB), lambda j: (0, j))]
                   + [cst((M, IMG))] * 8
                   + [cst((B, N_ALL))] + [cst((M, 1))] * 4),
        out_shape=([jax.ShapeDtypeStruct((M, OUTD), jnp.float32)]
                   + [tabi, tabi, tabf, tabf, tabi, tabi, tabf, tabf]
                   + [jax.ShapeDtypeStruct((B, N_ALL), jnp.float32)]
                   + [coli] * 4),
        scratch_shapes=[pltpu.VMEM((M, HID), jnp.float32)],
    )(zf, W1, b1.reshape(1, HID), W2, b2.reshape(1, OUTD), lin, zw_f, ld)

    meta = jnp.concatenate(
        [ylo, yhi, xblo, xbhi, jnp.zeros((M, 12), jnp.int32)], axis=1)
    meta = meta.reshape(M * 16)
    wv = jnp.broadcast_to(wmat.reshape(M, 1), (M, 16)).reshape(M * 16)
    zz = jnp.zeros((3 * _CHAN,), jnp.float32)
    d3f = decoded.reshape(M * 3 * _PATCH)

    tabs = [t.reshape(MI) for t in (x0, x1, wx0, wx1, y0, y1, wy0, wy1)]

    mesh = plsc.VectorSubcoreMesh(core_axis_name="c", subcore_axis_name="s")
    out5 = pl.kernel(
        _sc_merge_body,
        out_type=jax.ShapeDtypeStruct((B, 3, BANDS, _CHAN), jnp.float32),
        mesh=mesh,
        scratch_types=(
            [pltpu.VMEM((MI // B,), jnp.int32)] * 2
            + [pltpu.VMEM((MI // B,), jnp.float32)] * 2
            + [pltpu.VMEM((MI // B,), jnp.int32)] * 2
            + [pltpu.VMEM((MI // B,), jnp.float32)] * 2
            + [pltpu.VMEM((N_ALL * 16,), jnp.int32),
               pltpu.VMEM((N_ALL * 16,), jnp.float32),
               pltpu.VMEM((2 * 3 * _PATCH,), jnp.float32),
               pltpu.VMEM((3 * _CHAN,), jnp.float32),
               pltpu.VMEM((3 * _CHAN,), jnp.float32),
               pltpu.SemaphoreType.DMA,
               pltpu.SemaphoreType.DMA]
        ),
        compiler_params=pltpu.CompilerParams(needs_layout_passes=False),
    )(d3f, *tabs, meta, wv, zz)

    return out5.reshape(B, 3, IMG, IMG)


# speedup vs baseline: 1.0920x; 1.0180x over previous
import functools

import jax
import jax.numpy as jnp
from jax import lax
from jax.experimental import pallas as pl
from jax.experimental.pallas import tpu as pltpu
from jax.experimental.pallas import tpu_sc as plsc

B = 4
N_OBJ = 31
N_ALL = N_OBJ + 1
Z_WHAT = 64
IMG = 224
OBJ = 64
HID = 1024
OUTD = 3 * OBJ * OBJ
NB = 1536
HALF = (OBJ - 1) / 2.0
BANDS = 8
BROWS = IMG // BANDS
XBLK = IMG // 16


def _axis_tables(lin, ctr, scale):
    s = jnp.maximum(scale, 1e-6)
    p = ((lin - (2.0 * ctr - 1.0)) / s + 1.0) * HALF
    p0 = jnp.floor(p)
    w = p - p0
    v0 = (p0 >= 0) & (p0 <= OBJ - 1)
    v1 = (p0 + 1.0 >= 0) & (p0 + 1.0 <= OBJ - 1)
    i0 = jnp.clip(p0, 0, OBJ - 1).astype(jnp.int32)
    i1 = jnp.clip(p0 + 1.0, 0, OBJ - 1).astype(jnp.int32)
    w0 = jnp.where(v0, 1.0 - w, 0.0)
    w1 = jnp.where(v1, w, 0.0)
    return i0, i1, w0, w1


def _decode_body(z_ref, W1_ref, b1_ref, W2_ref, b2_ref, lin_ref, zw_ref,
                 ld_ref, out_ref,
                 x0_ref, wx0_ref, wx1_ref,
                 y0_ref, wy0_ref, wy1_ref,
                 w_ref, ylo_ref, yhi_ref, xblo_ref, xbhi_ref, h_ref):
    j = pl.program_id(0)

    @pl.when(j == 0)
    def _():
        h = jnp.dot(z_ref[...], W1_ref[...], preferred_element_type=jnp.float32)
        h_ref[...] = jnp.maximum(h + b1_ref[...], 0.0)

        lin = lin_ref[...]
        x0, _, wx0, wx1 = _axis_tables(lin, zw_ref[:, 0:1], zw_ref[:, 2:3])
        y0, _, wy0, wy1 = _axis_tables(lin, zw_ref[:, 1:2], zw_ref[:, 3:4])
        x0_ref[...] = x0
        wx0_ref[...] = wx0
        wx1_ref[...] = wx1
        y0_ref[...] = y0 * OBJ
        wy0_ref[...] = wy0
        wy1_ref[...] = wy1

        ld = ld_ref[...]
        m = jnp.max(ld, axis=1, keepdims=True)
        e = jnp.exp(ld - m)
        s = jnp.sum(e, axis=1, keepdims=True)
        wmat = e / s
        col = jax.lax.broadcasted_iota(jnp.int32, (B, N_ALL), 1)
        w_ref[...] = jnp.where(col == N_OBJ, 1.0, wmat)

        t = jax.lax.broadcasted_iota(jnp.int32, (B * N_ALL, IMG), 1)
        vy = (wy0 != 0.0) | (wy1 != 0.0)
        ylo_ref[...] = jnp.min(jnp.where(vy, t, IMG), axis=1, keepdims=True)
        yhi_ref[...] = jnp.max(jnp.where(vy, t + 1, 0), axis=1, keepdims=True)
        vx = (wx0 != 0.0) | (wx1 != 0.0)
        xlo = jnp.min(jnp.where(vx, t, IMG), axis=1, keepdims=True)
        xhi = jnp.max(jnp.where(vx, t + 1, 0), axis=1, keepdims=True)
        xblo_ref[...] = xlo // 16
        xbhi_ref[...] = (xhi + 15) // 16

    y = jnp.dot(h_ref[...], W2_ref[...], preferred_element_type=jnp.float32)
    out_ref[...] = jax.nn.sigmoid(y + b2_ref[...])


_PATCH = OBJ * OBJ
_CHAN = BROWS * IMG


def _sc_merge_body(d3, x0_h, wx0_h, wx1_h, y0_h, wy0_h, wy1_h,
                   meta_h, w_h, zz_h, out_h,
                   x0_v, wx0_v, wx1_v, y0_v, wy0_v, wy1_v,
                   meta_v, w_v, patch_v, acc_v, bg_v, sem0, sem1):
    cc = lax.axis_index("c")
    ss = lax.axis_index("s")
    wid = ss * 2 + cc
    b = wid // BANDS
    band = wid % BANDS
    blo = band * BROWS
    row0 = b * N_ALL

    t0 = row0 * IMG
    for hsrc, vdst in ((x0_h, x0_v), (wx0_h, wx0_v), (wx1_h, wx1_v),
                       (y0_h, y0_v), (wy0_h, wy0_v), (wy1_h, wy1_v)):
        pltpu.sync_copy(hsrc.at[pl.ds(t0, N_ALL * IMG)], vdst)
    pltpu.sync_copy(meta_h.at[pl.ds(row0 * 16, N_ALL * 16)], meta_v)
    pltpu.sync_copy(w_h.at[pl.ds(row0 * 16, N_ALL * 16)], w_v)
    pltpu.sync_copy(zz_h, acc_v)
    pltpu.sync_copy(zz_h, bg_v)

    def spl(v):
        return jnp.full((16,), v, jnp.int32)

    PB = 3 * _PATCH

    def start_fetch(i, buf, sem):
        pltpu.make_async_copy(
            d3.at[pl.ds((row0 + i) * PB, PB)],
            patch_v.at[pl.ds(buf * PB, PB)], sem).start()

    def wait_fetch(buf, sem):
        pltpu.make_async_copy(
            d3.at[pl.ds(0, PB)],
            patch_v.at[pl.ds(buf * PB, PB)], sem).wait()

    def do_object(i, boff, tgt):
        mrow = meta_v[pl.ds(i * 16, 16)]
        ylo = jnp.maximum(mrow[0], blo)
        yhi = jnp.minimum(mrow[1], blo + BROWS)
        xblo = mrow[2]
        xbhi = mrow[3]
        ws = w_v[pl.ds(i * 16, 16)][0]
        toff = i * IMG

        @pl.when((ylo < yhi) & (ws != 0.0))
        def _():
            @plsc.parallel_loop(ylo, yhi)
            def row_body(y):
                sy = spl(toff + y)
                y0m = plsc.load_gather(y0_v, [sy])
                y1m = jnp.minimum(y0m + OBJ, (OBJ - 1) * OBJ)
                wy0 = plsc.load_gather(wy0_v, [sy]) * ws
                wy1 = plsc.load_gather(wy1_v, [sy]) * ws
                aoff0 = (y - blo) * IMG

                @plsc.parallel_loop(xblo, xbhi, unroll=4)
                def col_body(xb):
                    bs = xb * 16
                    x0v = x0_v[pl.ds(toff + bs, 16)]
                    x1v = jnp.minimum(x0v + 1, OBJ - 1)
                    wx0v = wx0_v[pl.ds(toff + bs, 16)]
                    wx1v = wx1_v[pl.ds(toff + bs, 16)]
                    for c in range(3):
                        co = boff + c * _PATCH
                        g00 = plsc.load_gather(patch_v, [y0m + x0v + co])
                        g01 = plsc.load_gather(patch_v, [y0m + x1v + co])
                        g10 = plsc.load_gather(patch_v, [y1m + x0v + co])
                        g11 = plsc.load_gather(patch_v, [y1m + x1v + co])
                        r = (wy0 * (wx0v * g00 + wx1v * g01)
                             + wy1 * (wx0v * g10 + wx1v * g11))
                        aoff = c * _CHAN + aoff0 + bs
                        cur = tgt[pl.ds(aoff, 16)]
                        tgt[pl.ds(aoff, 16)] = cur + r

    start_fetch(0, 0, sem0)
    start_fetch(1, 1, sem1)

    def pair_body(j, _):
        i0 = j * 2
        wait_fetch(0, sem0)
        do_object(i0, 0, acc_v)

        @pl.when(i0 + 2 < N_ALL)
        def _():
            start_fetch(i0 + 2, 0, sem0)

        wait_fetch(1, sem1)
        do_object(i0 + 1, PB, acc_v)

        @pl.when(i0 + 3 < N_ALL)
        def _():
            start_fetch(i0 + 3, 1, sem1)

        return 0

    lax.fori_loop(0, N_ALL // 2 - 1, pair_body, 0)
    wait_fetch(0, sem0)
    do_object(N_ALL - 2, 0, acc_v)
    wait_fetch(1, sem1)
    do_object(N_ALL - 1, PB, bg_v)

    @plsc.parallel_loop(0, 3 * _CHAN // 16, unroll=4)
    def crow(j):
        off = j * 16
        a = acc_v[pl.ds(off, 16)]
        g = bg_v[pl.ds(off, 16)]
        acc_v[pl.ds(off, 16)] = a + g * jnp.where(a < 0.001, 1.0, 0.0)
    for c in range(3):
        pltpu.sync_copy(acc_v.at[pl.ds(c * _CHAN, _CHAN)], out_h.at[b, c, band])


@jax.jit
def kernel(z_what, z_where, z_present, z_depth, W1, b1, W2, b2):
    zf = z_what.reshape(B * N_ALL, Z_WHAT)

    M = B * N_ALL
    MI = M * IMG

    bg_where = jnp.broadcast_to(
        jnp.array([0.5, 0.5, 1.0, 1.0], jnp.float32), (B, 1, 4))
    zw_f = jnp.concatenate([z_where, bg_where], axis=1).reshape(M, 4)

    neg_inf = jnp.full((B, 1), -jnp.inf, jnp.float32)
    dcol = jnp.concatenate([z_depth[..., 0], neg_inf], axis=1)
    pcol = jnp.concatenate([z_present[..., 0], jnp.zeros((B, 1))], axis=1)
    ld = jnp.where(pcol == 1.0, dcol, -jnp.inf)

    lin = jnp.linspace(-1.0, 1.0, IMG).astype(jnp.float32).reshape(1, IMG)

    tabf = jax.ShapeDtypeStruct((M, IMG), jnp.float32)
    tabi = jax.ShapeDtypeStruct((M, IMG), jnp.int32)
    coli = jax.ShapeDtypeStruct((M, 1), jnp.int32)

    cst = lambda s: pl.BlockSpec(s, lambda j: tuple(0 for _ in s))
    (decoded, x0, wx0, wx1, y0, wy0, wy1,
     wmat, ylo, yhi, xblo, xbhi) = pl.pallas_call(
        _decode_body,
        grid=(OUTD // NB,),
        in_specs=[
            cst((M, Z_WHAT)),
            cst((Z_WHAT, HID)),
            cst((1, HID)),
            pl.BlockSpec((HID, NB), lambda j: (0, j)),
            pl.BlockSpec((1, NB), lambda j: (0, j)),
            cst((1, IMG)),
            cst((M, 4)),
            cst((B, N_ALL)),
        ],
        out_specs=([pl.BlockSpec((M, NB), lambda j: (0, j))]
                   + [cst((M, IMG))] * 6
                   + [cst((B, N_ALL))] + [cst((M, 1))] * 4),
        out_shape=([jax.ShapeDtypeStruct((M, OUTD), jnp.float32)]
                   + [tabi, tabf, tabf, tabi, tabf, tabf]
                   + [jax.ShapeDtypeStruct((B, N_ALL), jnp.float32)]
                   + [coli] * 4),
        scratch_shapes=[pltpu.VMEM((M, HID), jnp.float32)],
    )(zf, W1, b1.reshape(1, HID), W2, b2.reshape(1, OUTD), lin, zw_f, ld)

    meta = jnp.concatenate(
        [ylo, yhi, xblo, xbhi, jnp.zeros((M, 12), jnp.int32)], axis=1)
    meta = meta.reshape(M * 16)
    wv = jnp.broadcast_to(wmat.reshape(M, 1), (M, 16)).reshape(M * 16)
    zz = jnp.zeros((3 * _CHAN,), jnp.float32)
    d3f = decoded.reshape(M * 3 * _PATCH)

    tabs = [t.reshape(MI) for t in (x0, wx0, wx1, y0, wy0, wy1)]

    mesh = plsc.VectorSubcoreMesh(core_axis_name="c", subcore_axis_name="s")
    out5 = pl.kernel(
        _sc_merge_body,
        out_type=jax.ShapeDtypeStruct((B, 3, BANDS, _CHAN), jnp.float32),
        mesh=mesh,
        scratch_types=(
            [pltpu.VMEM((MI // B,), jnp.int32)]
            + [pltpu.VMEM((MI // B,), jnp.float32)] * 2
            + [pltpu.VMEM((MI // B,), jnp.int32)]
            + [pltpu.VMEM((MI // B,), jnp.float32)] * 2
            + [pltpu.VMEM((N_ALL * 16,), jnp.int32),
               pltpu.VMEM((N_ALL * 16,), jnp.float32),
               pltpu.VMEM((2 * 3 * _PATCH,), jnp.float32),
               pltpu.VMEM((3 * _CHAN,), jnp.float32),
               pltpu.VMEM((3 * _CHAN,), jnp.float32),
               pltpu.SemaphoreType.DMA,
               pltpu.SemaphoreType.DMA]
        ),
        compiler_params=pltpu.CompilerParams(needs_layout_passes=False),
    )(d3f, *tabs, meta, wv, zz)

    return out5.reshape(B, 3, IMG, IMG)
